# baseline (speedup 1.0000x reference)
import jax
import jax.numpy as jnp
from jax.experimental import pallas as pl
from jax.experimental.pallas import tpu as pltpu


def _vote_kernel(x_ref, n_ref, o_ref):
    x = x_ref[...]
    w = 1.0 + n_ref[...] * 0.001
    x0, x1, x2, x3, x4 = x[0], x[1], x[2], x[3], x[4]
    p1 = jnp.maximum(x0, x1)
    p2 = jnp.maximum(p1, x2)
    p3 = jnp.maximum(p2, x3)
    m = jnp.maximum(p3, x4)
    v0 = x0 == m
    v1 = (x1 == m) & (x0 < m)
    v2 = (x2 == m) & (p1 < m)
    v3 = (x3 == m) & (p2 < m)
    v4 = (x4 == m) & (p3 < m)
    zero = jnp.zeros_like(w)
    o_ref[0, :] = jnp.sum(jnp.where(v0, w, zero), axis=0)
    o_ref[1, :] = jnp.sum(jnp.where(v1, w, zero), axis=0)
    o_ref[2, :] = jnp.sum(jnp.where(v2, w, zero), axis=0)
    o_ref[3, :] = jnp.sum(jnp.where(v3, w, zero), axis=0)
    o_ref[4, :] = jnp.sum(jnp.where(v4, w, zero), axis=0)


def kernel(expert_logits, noise):
    E, B, S, C = expert_logits.shape
    N = B * S
    LB = 12800
    xt = jnp.transpose(expert_logits.reshape(E, N, C), (2, 0, 1))
    nz = noise.reshape(E, N)

    out_t = pl.pallas_call(
        _vote_kernel,
        grid=(N // LB,),
        in_specs=[
            pl.BlockSpec((C, E, LB), lambda i: (0, 0, i)),
            pl.BlockSpec((E, LB), lambda i: (0, i)),
        ],
        out_specs=pl.BlockSpec((C, LB), lambda i: (0, i)),
        out_shape=jax.ShapeDtypeStruct((C, N), expert_logits.dtype),
        compiler_params=pltpu.CompilerParams(
            dimension_semantics=("parallel",),
        ),
    )(xt, nz)
    return jnp.transpose(out_t, (1, 0)).reshape(B, S, C)
